# Initial kernel scaffold; baseline (speedup 1.0000x reference)
import jax
import jax.numpy as jnp
from jax.experimental import pallas as pl


def kernel(coords, elements, nodal_values):
    raise NotImplementedError("write your pallas kernel here")


# speedup vs baseline: 16.4101x; 16.4101x over previous
import functools

import jax
import jax.numpy as jnp
from jax import lax
from jax.experimental import pallas as pl
from jax.experimental.pallas import tpu as pltpu
from jax.experimental.pallas import tpu_sc as plsc

N_NODES = 100000
N_ELEMENTS = 200000
D_FEAT = 128

NC, NS, L = 2, 16, 16
NW = NC * NS
K = 128
NCH = 50
EW = NCH * K
E_PAD = NW * EW
SLICE = 6256
N_PAD = NS * SLICE

_mesh = plsc.VectorSubcoreMesh(
    core_axis_name="c", subcore_axis_name="s", num_cores=NC, num_subcores=NS
)


@functools.partial(
    pl.kernel,
    out_type=(
        jax.ShapeDtypeStruct((N_PAD,), jnp.float32),
        jax.ShapeDtypeStruct((N_PAD,), jnp.float32),
    ),
    mesh=_mesh,
    scratch_types=[
        pltpu.VMEM((NCH, K), jnp.int32),
        pltpu.VMEM((NCH, K), jnp.int32),
        pltpu.VMEM((NCH, K), jnp.int32),
        pltpu.VMEM((NCH, K), jnp.float32),
        pltpu.VMEM((NCH, K), jnp.float32),
        pltpu.VMEM((NCH, K), jnp.float32),
        pltpu.VMEM((NCH, K), jnp.float32),
        pltpu.VMEM((NCH, K), jnp.float32),
        pltpu.VMEM((NCH, K), jnp.float32),
        pltpu.VMEM((NCH, K), jnp.float32),
        pltpu.VMEM((SLICE,), jnp.float32),
        pltpu.VMEM_SHARED((N_PAD,), jnp.float32),
        pltpu.SemaphoreType.DMA,
    ],
)
def _sc_node_weights(xs_hbm, ys_hbm, i0_hbm, i1_hbm, i2_hbm,
                     out0_hbm, out1_hbm,
                     i0_v, i1_v, i2_v, x0_v, x1_v, x2_v, y0_v, y1_v, y2_v,
                     w_v, z_v, s_sh, sem):
    cid = lax.axis_index("c")
    sid = lax.axis_index("s")
    wid = sid * NC + cid

    def _zero(j, carry):
        z_v[pl.ds(j * L, L)] = jnp.zeros((L,), jnp.float32)
        return carry

    lax.fori_loop(0, SLICE // L, _zero, 0)
    pltpu.sync_copy(z_v, s_sh.at[pl.ds(sid * SLICE, SLICE)])

    c0 = pltpu.async_copy(i0_hbm.at[wid], i0_v, sem)
    c1 = pltpu.async_copy(i1_hbm.at[wid], i1_v, sem)
    c2 = pltpu.async_copy(i2_hbm.at[wid], i2_v, sem)
    c0.wait()
    c1.wait()
    c2.wait()

    plsc.subcore_barrier()

    def _chunk(j, carry):
        r0, r1, r2 = i0_v.at[j], i1_v.at[j], i2_v.at[j]
        g0 = pltpu.async_copy(xs_hbm.at[r0], x0_v.at[j], sem)
        g1 = pltpu.async_copy(xs_hbm.at[r1], x1_v.at[j], sem)
        g2 = pltpu.async_copy(xs_hbm.at[r2], x2_v.at[j], sem)
        g3 = pltpu.async_copy(ys_hbm.at[r0], y0_v.at[j], sem)
        g4 = pltpu.async_copy(ys_hbm.at[r1], y1_v.at[j], sem)
        g5 = pltpu.async_copy(ys_hbm.at[r2], y2_v.at[j], sem)
        g0.wait(); g1.wait(); g2.wait(); g3.wait(); g4.wait(); g5.wait()
        for k in range(K // L):
            sl = pl.ds(k * L, L)
            ax = x1_v[j, sl] - x0_v[j, sl]
            ay = y1_v[j, sl] - y0_v[j, sl]
            bx = x2_v[j, sl] - x0_v[j, sl]
            by = y2_v[j, sl] - y0_v[j, sl]
            det = ax * by - bx * ay
            w_v[j, sl] = jnp.abs(det) * jnp.float32(1.0 / 6.0)
        wrow = w_v.at[j]
        pltpu.sync_copy(wrow, s_sh.at[r0], add=True)
        pltpu.sync_copy(wrow, s_sh.at[r1], add=True)
        pltpu.sync_copy(wrow, s_sh.at[r2], add=True)
        return carry

    lax.fori_loop(0, NCH, _chunk, 0)

    plsc.subcore_barrier()
    sl_out = pl.ds(sid * SLICE, SLICE)
    pltpu.sync_copy(s_sh.at[sl_out], z_v)

    @pl.when(cid == 0)
    def _():
        pltpu.sync_copy(z_v, out0_hbm.at[sl_out])

    @pl.when(cid == 1)
    def _():
        pltpu.sync_copy(z_v, out1_hbm.at[sl_out])


_BN = 4000


def _tc_body(s0_ref, s1_ref, v_ref, o_ref):
    i = pl.program_id(0)

    @pl.when(i == 0)
    def _():
        o_ref[...] = jnp.zeros_like(o_ref)

    s = s0_ref[...] + s1_ref[...]
    o_ref[...] += jnp.sum(v_ref[...] * s, axis=0, keepdims=True)


def kernel(coords, elements, nodal_values):
    el = elements.astype(jnp.int32)
    xs = jnp.zeros((N_PAD,), jnp.float32).at[:N_NODES].set(coords[:, 0])
    ys = jnp.zeros((N_PAD,), jnp.float32).at[:N_NODES].set(coords[:, 1])
    idx = jnp.full((3, E_PAD), N_NODES, jnp.int32).at[:, :N_ELEMENTS].set(el.T)
    idx = idx.reshape(3, NW, NCH, K)

    sa, sb = _sc_node_weights(xs, ys, idx[0], idx[1], idx[2])

    s0 = sa.reshape(N_PAD, 1)
    s1 = sb.reshape(N_PAD, 1)
    out = pl.pallas_call(
        _tc_body,
        grid=(N_NODES // _BN,),
        in_specs=[
            pl.BlockSpec((_BN, 1), lambda i: (i, 0)),
            pl.BlockSpec((_BN, 1), lambda i: (i, 0)),
            pl.BlockSpec((_BN, D_FEAT), lambda i: (i, 0)),
        ],
        out_specs=pl.BlockSpec((1, D_FEAT), lambda i: (0, 0)),
        out_shape=jax.ShapeDtypeStruct((1, D_FEAT), jnp.float32),
    )(s0, s1, nodal_values)
    return out[0]
